# baseline (speedup 1.0000x reference)
import functools

import jax
import jax.numpy as jnp
from jax.experimental import pallas as pl
from jax.experimental.pallas import tpu as pltpu

_BLOCK_ROWS = 1024


def _gat_body(x_ref, adjl_ref, adjr_ref, wt_ref, a_ref, out_ref, u_ref, v_ref,
              tp_ref, t2_ref, whx_ref, *, fout):
    i = pl.program_id(0)
    n = x_ref.shape[0]
    c = jnp.float32(1.4426950408889634)

    @pl.when(i == 0)
    def _init():
        wh = jax.lax.dot_general(
            x_ref[...], wt_ref[...], (((1,), (1,)), ((), ())),
            preferred_element_type=jnp.float32)
        a1 = a_ref[:, :fout]
        a2 = a_ref[:, fout:]
        tp = c * jax.lax.dot_general(
            a2, wh, (((1,), (1,)), ((), ())), preferred_element_type=jnp.float32
        )
        tp_ref[...] = tp
        t2_ref[...] = 0.2 * tp
        sp = c * jax.lax.dot_general(
            wh, a1, (((1,), (1,)), ((), ())), preferred_element_type=jnp.float32
        )
        mx = sp + jnp.max(tp)
        mp = jnp.maximum(mx, 0.2 * mx)
        u_ref[...] = sp - mp
        v_ref[...] = 0.2 * sp - mp
        ext = jnp.concatenate(
            [wh, jnp.ones((n, 1), jnp.float32),
             jnp.zeros((n, 16 - 1), jnp.float32)], axis=1)
        whx_ref[...] = ext.astype(jnp.bfloat16)

    br = adjl_ref.shape[0]
    half = adjl_ref.shape[1]
    u = u_ref[pl.ds(i * br, br), :]
    v = v_ref[pl.ds(i * br, br), :]
    arg = jnp.maximum(u + tp_ref[:, :half], v + t2_ref[:, :half])
    p = (jnp.exp2(arg) * adjl_ref[...].astype(jnp.float32)).astype(jnp.bfloat16)
    rt = jax.lax.dot_general(
        whx_ref[:half, :], p, (((0,), (1,)), ((), ())),
        preferred_element_type=jnp.float32)
    arg = jnp.maximum(u + tp_ref[:, half:], v + t2_ref[:, half:])
    p = (jnp.exp2(arg) * adjr_ref[...].astype(jnp.float32)).astype(jnp.bfloat16)
    rt = rt + jax.lax.dot_general(
        whx_ref[half:, :], p, (((0,), (1,)), ((), ())),
        preferred_element_type=jnp.float32)
    acc = rt[:fout, :]
    l = rt[fout:fout + 1, :]
    h = acc / l
    out_ref[...] = jnp.where(h > 0, h, jnp.exp(h) - 1.0)


def kernel(input, adj, W, a):
    n, fin = input.shape
    fout = W.shape[1]
    br = _BLOCK_ROWS
    grid = (n // br,)
    out_t = pl.pallas_call(
        functools.partial(_gat_body, fout=fout),
        grid=grid,
        in_specs=[
            pl.BlockSpec((n, fin), lambda i: (0, 0)),
            pl.BlockSpec((br, n // 2), lambda i: (i, 0)),
            pl.BlockSpec((br, n // 2), lambda i: (i, 1)),
            pl.BlockSpec((fout, fin), lambda i: (0, 0)),
            pl.BlockSpec((1, 2 * fout), lambda i: (0, 0)),
        ],
        out_specs=pl.BlockSpec((fout, br), lambda i: (0, i)),
        out_shape=jax.ShapeDtypeStruct((fout, n), jnp.float32),
        scratch_shapes=[
            pltpu.VMEM((n, 1), jnp.float32),
            pltpu.VMEM((n, 1), jnp.float32),
            pltpu.VMEM((1, n), jnp.float32),
            pltpu.VMEM((1, n), jnp.float32),
            pltpu.VMEM((n, fout + 16), jnp.bfloat16),
        ],
    )(input, adj, adj, W.T, a)
    return out_t.T


# speedup vs baseline: 1.1085x; 1.1085x over previous
import functools

import jax
import jax.numpy as jnp
from jax.experimental import pallas as pl
from jax.experimental.pallas import tpu as pltpu

_BLOCK_ROWS = 1024


def _gat_body(x_ref, adj_ref, wt_ref, a_ref, out_ref, u_ref, v_ref,
              tp_ref, t2_ref, whx_ref, *, fout):
    i = pl.program_id(0)
    n = x_ref.shape[0]
    c = jnp.float32(1.4426950408889634)

    @pl.when(i == 0)
    def _init():
        wh = jax.lax.dot_general(
            x_ref[...], wt_ref[...], (((1,), (1,)), ((), ())),
            preferred_element_type=jnp.float32)
        a1 = a_ref[:, :fout]
        a2 = a_ref[:, fout:]
        tp = c * jax.lax.dot_general(
            a2, wh, (((1,), (1,)), ((), ())), preferred_element_type=jnp.float32
        )
        tp_ref[...] = tp
        t2_ref[...] = 0.2 * tp
        sp = c * jax.lax.dot_general(
            wh, a1, (((1,), (1,)), ((), ())), preferred_element_type=jnp.float32
        )
        mx = sp + jnp.max(tp)
        mp = jnp.maximum(mx, 0.2 * mx)
        u_ref[...] = sp - mp
        v_ref[...] = 0.2 * sp - mp
        ext = jnp.concatenate(
            [wh, jnp.ones((n, 1), jnp.float32),
             jnp.zeros((n, 16 - 1), jnp.float32)], axis=1)
        whx_ref[...] = ext.astype(jnp.bfloat16)

    br = adj_ref.shape[0]
    u = u_ref[pl.ds(i * br, br), :]
    v = v_ref[pl.ds(i * br, br), :]
    arg = jnp.maximum(u + tp_ref[...], v + t2_ref[...])
    p = (jnp.exp2(arg) * adj_ref[...].astype(jnp.float32)).astype(jnp.bfloat16)
    rt = jax.lax.dot_general(
        whx_ref[...], p, (((0,), (1,)), ((), ())),
        preferred_element_type=jnp.float32)
    acc = rt[:fout, :]
    l = rt[fout:fout + 1, :]
    h = acc / l
    out_ref[...] = jnp.where(h > 0, h, jnp.exp(h) - 1.0)


def kernel(input, adj, W, a):
    n, fin = input.shape
    fout = W.shape[1]
    br = _BLOCK_ROWS
    grid = (n // br,)
    out_t = pl.pallas_call(
        functools.partial(_gat_body, fout=fout),
        grid=grid,
        in_specs=[
            pl.BlockSpec((n, fin), lambda i: (0, 0)),
            pl.BlockSpec((br, n), lambda i: (i, 0)),
            pl.BlockSpec((fout, fin), lambda i: (0, 0)),
            pl.BlockSpec((1, 2 * fout), lambda i: (0, 0)),
        ],
        out_specs=pl.BlockSpec((fout, br), lambda i: (0, i)),
        out_shape=jax.ShapeDtypeStruct((fout, n), jnp.float32),
        scratch_shapes=[
            pltpu.VMEM((n, 1), jnp.float32),
            pltpu.VMEM((n, 1), jnp.float32),
            pltpu.VMEM((1, n), jnp.float32),
            pltpu.VMEM((1, n), jnp.float32),
            pltpu.VMEM((n, fout + 16), jnp.bfloat16),
        ],
    )(input, adj, W.T, a)
    return out_t.T
